# Initial kernel scaffold; baseline (speedup 1.0000x reference)
import jax
import jax.numpy as jnp
from jax.experimental import pallas as pl


def kernel(POI_embs, delta_dis_embs, sess_idx, edge_index, edge_dist, attention_weight, alpha_src, alpha_dst):
    raise NotImplementedError("write your pallas kernel here")


# speedup vs baseline: 24.8350x; 24.8350x over previous
import functools

import jax
import jax.numpy as jnp
from jax import lax
from jax.experimental import pallas as pl
from jax.experimental.pallas import tpu as pltpu
from jax.experimental.pallas import tpu_sc as plsc

NC = 2
NS = 16
NW = NC * NS
L = 16

N = 10000
NP = 10240
D = 128
E = 320000
ND = 1024

RPT = NP // NW


def _vgather(v, idx):
    return lax.gather(
        v, idx[:, None],
        dimension_numbers=lax.GatherDimensionNumbers(
            offset_dims=(), collapsed_slice_dims=(0,), start_index_map=(0,)),
        slice_sizes=(1,),
        mode=lax.GatherScatterMode.PROMISE_IN_BOUNDS)


def _allsum(v):
    idx = lax.iota(jnp.int32, L)
    for sh in (1, 2, 4, 8):
        v = v + _vgather(v, idx ^ sh)
    return v


def _bcast_lane(v, jj):
    return _vgather(v, jnp.full((L,), jj, jnp.int32))
RC = 80
EPT = E // NS
C = 80
NCHUNK = EPT // C
RRT = NP // NS


def _scores_body(poi_hbm, sidx_hbm, delta_hbm, ws_hbm, wd_hbm,
                 x_hbm, es_hbm, edt_hbm, ed_hbm,
                 idx_v, rows_v, ws_v, wd_v, e_v, f_v,
                 drows_v, de_v, sem):
    wid = lax.axis_index("s") * NC + lax.axis_index("c")
    base = wid * RPT
    pltpu.sync_copy(ws_hbm, ws_v)
    pltpu.sync_copy(wd_hbm, wd_v)
    lane = lax.iota(jnp.int32, L)

    def chunk(k, _):
        cb = base + k * RC
        pltpu.sync_copy(sidx_hbm.at[pl.ds(cb, RC)], idx_v)
        pltpu.async_copy(poi_hbm.at[idx_v], rows_v, sem).wait()
        pltpu.sync_copy(rows_v, x_hbm.at[pl.ds(cb, RC)])

        def grp(g, _):
            sres = jnp.zeros((L,), jnp.float32)
            dres = jnp.zeros((L,), jnp.float32)
            for jj in range(L):
                j = g * L + jj
                acc_s = jnp.zeros((L,), jnp.float32)
                acc_d = jnp.zeros((L,), jnp.float32)
                for r in range(D // L):
                    xr = rows_v[j, pl.ds(r * L, L)]
                    acc_s = acc_s + xr * ws_v[pl.ds(r * L, L)]
                    acc_d = acc_d + xr * wd_v[pl.ds(r * L, L)]
                sres = jnp.where(lane == jj, _allsum(acc_s), sres)
                dres = jnp.where(lane == jj, _allsum(acc_d), dres)
            e_v[pl.ds(g * L, L)] = jnp.exp(sres)
            f_v[pl.ds(g * L, L)] = jnp.exp(dres)
            return _

        lax.fori_loop(0, RC // L, grp, None)
        pltpu.sync_copy(e_v, es_hbm.at[pl.ds(cb, RC)])
        pltpu.sync_copy(f_v, edt_hbm.at[pl.ds(cb, RC)])
        return _

    lax.fori_loop(0, RPT // RC, chunk, None)

    dpt = ND // NW
    dbase = wid * dpt
    pltpu.sync_copy(delta_hbm.at[pl.ds(dbase, dpt)], drows_v)

    def dgrp(g, _):
        dres = jnp.zeros((L,), jnp.float32)
        for jj in range(L):
            j = g * L + jj
            acc = jnp.zeros((L,), jnp.float32)
            for r in range(D // L):
                acc = acc + drows_v[j, pl.ds(r * L, L)] * ws_v[pl.ds(r * L, L)]
            dres = jnp.where(lane == jj, _allsum(acc), dres)
        de_v[pl.ds(g * L, L)] = jnp.exp(dres)
        return _

    lax.fori_loop(0, dpt // L, dgrp, None)
    pltpu.sync_copy(de_v, ed_hbm.at[pl.ds(dbase, dpt)])


def _edge_body(x_hbm, ei0_hbm, ei1_hbm, dist_hbm, ed_hbm,
               fb_hbm, pc_hbm,
               i0_v, i1_v, dist_v, gsel_v, ssel_v, w_v, rows_v,
               ed_v, sca_v, red_v, tmp_v, zrows_v, sem,
               acc_sh, stage_sh):
    cid = lax.axis_index("c")
    sid = lax.axis_index("s")
    is_f = cid == 0
    lane = lax.iota(jnp.int32, L)

    pltpu.sync_copy(ed_hbm, ed_v)

    def zrow(j, _):
        zr = zrows_v.at[j]
        for r in range(D // L):
            zr[pl.ds(r * L, L)] = jnp.zeros((L,), jnp.float32)
        return _

    lax.fori_loop(0, 64, zrow, None)
    for k in range(RRT // 64):
        pltpu.sync_copy(zrows_v, acc_sh.at[pl.ds(sid * RRT + k * 64, 64)])

    def zsca(g, _):
        sca_v[pl.ds(g * L, L)] = jnp.zeros((L,), jnp.float32)
        return _

    lax.fori_loop(0, NP // L, zsca, None)
    plsc.subcore_barrier()

    ebase = sid * EPT

    def chunk(i, _):
        cb = ebase + i * C
        pltpu.sync_copy(ei0_hbm.at[pl.ds(cb, C)], i0_v)
        pltpu.sync_copy(ei1_hbm.at[pl.ds(cb, C)], i1_v)
        pltpu.sync_copy(dist_hbm.at[pl.ds(cb, C)], dist_v)

        def sel(g, _):
            sl = pl.ds(g * L, L)
            a = i0_v[sl]
            b = i1_v[sl]
            gsel_v[sl] = jnp.where(is_f, a, b)
            ssel_v[sl] = jnp.where(is_f, b, a)
            edl = plsc.load_gather(ed_v, [dist_v[sl]])
            w_v[sl] = jnp.where(is_f, edl, jnp.ones((L,), jnp.float32))
            return _

        lax.fori_loop(0, C // L, sel, None)

        pltpu.async_copy(x_hbm.at[gsel_v], rows_v, sem).wait()

        def scale(g, _):
            wl = w_v[pl.ds(g * L, L)]
            for jj in range(L):
                j = g * L + jj
                w = _bcast_lane(wl, jj)
                rr = rows_v.at[j]
                for r in range(D // L):
                    sl = pl.ds(r * L, L)
                    rr[sl] = rr[sl] * w
            return _

        lax.fori_loop(0, C // L, scale, None)

        pltpu.sync_copy(rows_v, acc_sh.at[ssel_v], add=True)

        def sacc(g, _):
            sl = pl.ds(g * L, L)
            plsc.addupdate_scatter(sca_v, [ssel_v[sl]], w_v[sl])
            return _

        lax.fori_loop(0, C // L, sacc, None)
        return _

    lax.fori_loop(0, NCHUNK, chunk, None)
    plsc.subcore_barrier()

    for k in range(RRT // C):
        rb = sid * RRT + k * C
        pltpu.sync_copy(acc_sh.at[pl.ds(rb, C)], rows_v)
        pltpu.sync_copy(rows_v, fb_hbm.at[cid, pl.ds(rb, C)])

    pltpu.sync_copy(sca_v, stage_sh.at[sid])
    plsc.subcore_barrier()
    rb = sid * RRT
    pltpu.sync_copy(stage_sh.at[0, pl.ds(rb, RRT)], red_v)

    def redt(t, _):
        pltpu.sync_copy(stage_sh.at[t, pl.ds(rb, RRT)], tmp_v)

        def add(g, _):
            sl = pl.ds(g * L, L)
            red_v[sl] = red_v[sl] + tmp_v[sl]
            return _

        lax.fori_loop(0, RRT // L, add, None)
        return _

    lax.fori_loop(1, NS, redt, None)
    pltpu.sync_copy(red_v, pc_hbm.at[cid, pl.ds(rb, RRT)])


@functools.partial(
    pl.kernel,
    out_type=(
        jax.ShapeDtypeStruct((NP, D), jnp.float32),
        jax.ShapeDtypeStruct((NP,), jnp.float32),
        jax.ShapeDtypeStruct((NP,), jnp.float32),
        jax.ShapeDtypeStruct((ND,), jnp.float32),
    ),
    mesh=plsc.VectorSubcoreMesh(core_axis_name="c", subcore_axis_name="s"),
    scratch_types=[
        pltpu.VMEM((RC,), jnp.int32),
        pltpu.VMEM((RC, D), jnp.float32),
        pltpu.VMEM((D,), jnp.float32),
        pltpu.VMEM((D,), jnp.float32),
        pltpu.VMEM((RC,), jnp.float32),
        pltpu.VMEM((RC,), jnp.float32),
        pltpu.VMEM((ND // NW, D), jnp.float32),
        pltpu.VMEM((ND // NW,), jnp.float32),
        pltpu.SemaphoreType.DMA,
    ],
)
def _scores_kernel(*refs):
    _scores_body(*refs)


@functools.partial(
    pl.kernel,
    out_type=(
        jax.ShapeDtypeStruct((NC, NP, D), jnp.float32),
        jax.ShapeDtypeStruct((NC, NP), jnp.float32),
    ),
    mesh=plsc.VectorSubcoreMesh(core_axis_name="c", subcore_axis_name="s"),
    scratch_types=[
        pltpu.VMEM((C,), jnp.int32),
        pltpu.VMEM((C,), jnp.int32),
        pltpu.VMEM((C,), jnp.int32),
        pltpu.VMEM((C,), jnp.int32),
        pltpu.VMEM((C,), jnp.int32),
        pltpu.VMEM((C,), jnp.float32),
        pltpu.VMEM((C, D), jnp.float32),
        pltpu.VMEM((ND,), jnp.float32),
        pltpu.VMEM((NP,), jnp.float32),
        pltpu.VMEM((RRT,), jnp.float32),
        pltpu.VMEM((RRT,), jnp.float32),
        pltpu.VMEM((64, D), jnp.float32),
        pltpu.SemaphoreType.DMA,
        pltpu.VMEM_SHARED((NP, D), jnp.float32),
        pltpu.VMEM_SHARED((NS, NP), jnp.float32),
    ],
    compiler_params=pltpu.CompilerParams(needs_layout_passes=False),
)
def _edge_kernel(*refs):
    _edge_body(*refs)


def _combine_body(f_ref, b_ref, es_ref, edt_ref, p_ref, c_ref, o_ref):
    es = es_ref[...]
    edt = edt_ref[...]
    denom = es * p_ref[...] + edt * c_ref[...] + 1e-16
    o_ref[...] = (es * f_ref[...] + edt * b_ref[...]) / denom


_combine = pl.pallas_call(
    _combine_body,
    grid=(10,),
    in_specs=[
        pl.BlockSpec((NP // 10, D), lambda i: (i, 0)),
        pl.BlockSpec((NP // 10, D), lambda i: (i, 0)),
        pl.BlockSpec((NP // 10, 1), lambda i: (i, 0)),
        pl.BlockSpec((NP // 10, 1), lambda i: (i, 0)),
        pl.BlockSpec((NP // 10, 1), lambda i: (i, 0)),
        pl.BlockSpec((NP // 10, 1), lambda i: (i, 0)),
    ],
    out_specs=pl.BlockSpec((NP // 10, D), lambda i: (i, 0)),
    out_shape=jax.ShapeDtypeStruct((NP, D), jnp.float32),
)


@jax.jit
def kernel(POI_embs, delta_dis_embs, sess_idx, edge_index, edge_dist,
           attention_weight, alpha_src, alpha_dst):
    w_src = attention_weight.T @ alpha_src
    w_dst = attention_weight.T @ alpha_dst
    sidx = jnp.concatenate(
        [sess_idx, jnp.zeros((NP - N,), jnp.int32)])
    x, es, edt, ed = _scores_kernel(
        POI_embs, sidx, delta_dis_embs, w_src, w_dst)
    fb, pc = _edge_kernel(
        x, edge_index[0], edge_index[1], edge_dist, ed)
    h = _combine(fb[0], fb[1], es[:, None], edt[:, None],
                 pc[0][:, None], pc[1][:, None])
    return h[:N]


# speedup vs baseline: 50.2721x; 2.0242x over previous
import functools

import jax
import jax.numpy as jnp
from jax import lax
from jax.experimental import pallas as pl
from jax.experimental.pallas import tpu as pltpu
from jax.experimental.pallas import tpu_sc as plsc

NC = 2
NS = 16
NW = NC * NS
L = 16

N = 10000
NP = 10240
D = 128
E = 320000
ND = 1024

RPT = NP // NW


def _vgather(v, idx):
    return lax.gather(
        v, idx[:, None],
        dimension_numbers=lax.GatherDimensionNumbers(
            offset_dims=(), collapsed_slice_dims=(0,), start_index_map=(0,)),
        slice_sizes=(1,),
        mode=lax.GatherScatterMode.PROMISE_IN_BOUNDS)


def _allsum(v):
    idx = lax.iota(jnp.int32, L)
    for sh in (1, 2, 4, 8):
        v = v + _vgather(v, idx ^ sh)
    return v


def _bcast_lane(v, jj):
    return _vgather(v, jnp.full((L,), jj, jnp.int32))
RC = 80
EPT = E // NS
C = 80
NCHUNK = EPT // C
RRT = NP // NS


def _scores_body(poi_hbm, sidx_hbm, delta_hbm, ws_hbm, wd_hbm,
                 x_hbm, es_hbm, edt_hbm, ed_hbm,
                 idx_v, rows_v, ws_v, wd_v, e_v, f_v,
                 drows_v, de_v, sem):
    wid = lax.axis_index("s") * NC + lax.axis_index("c")
    base = wid * RPT
    pltpu.sync_copy(ws_hbm, ws_v)
    pltpu.sync_copy(wd_hbm, wd_v)
    lane = lax.iota(jnp.int32, L)

    def chunk(k, _):
        cb = base + k * RC
        pltpu.sync_copy(sidx_hbm.at[pl.ds(cb, RC)], idx_v)
        pltpu.async_copy(poi_hbm.at[idx_v], rows_v, sem).wait()
        pltpu.sync_copy(rows_v, x_hbm.at[pl.ds(cb, RC)])

        def grp(g, _):
            sres = jnp.zeros((L,), jnp.float32)
            dres = jnp.zeros((L,), jnp.float32)
            for jj in range(L):
                j = g * L + jj
                acc_s = jnp.zeros((L,), jnp.float32)
                acc_d = jnp.zeros((L,), jnp.float32)
                for r in range(D // L):
                    xr = rows_v[j, pl.ds(r * L, L)]
                    acc_s = acc_s + xr * ws_v[pl.ds(r * L, L)]
                    acc_d = acc_d + xr * wd_v[pl.ds(r * L, L)]
                sres = jnp.where(lane == jj, _allsum(acc_s), sres)
                dres = jnp.where(lane == jj, _allsum(acc_d), dres)
            e_v[pl.ds(g * L, L)] = jnp.exp(sres)
            f_v[pl.ds(g * L, L)] = jnp.exp(dres)
            return _

        lax.fori_loop(0, RC // L, grp, None)
        pltpu.sync_copy(e_v, es_hbm.at[pl.ds(cb, RC)])
        pltpu.sync_copy(f_v, edt_hbm.at[pl.ds(cb, RC)])
        return _

    lax.fori_loop(0, RPT // RC, chunk, None)

    dpt = ND // NW
    dbase = wid * dpt
    pltpu.sync_copy(delta_hbm.at[pl.ds(dbase, dpt)], drows_v)

    def dgrp(g, _):
        dres = jnp.zeros((L,), jnp.float32)
        for jj in range(L):
            j = g * L + jj
            acc = jnp.zeros((L,), jnp.float32)
            for r in range(D // L):
                acc = acc + drows_v[j, pl.ds(r * L, L)] * ws_v[pl.ds(r * L, L)]
            dres = jnp.where(lane == jj, _allsum(acc), dres)
        de_v[pl.ds(g * L, L)] = jnp.exp(dres)
        return _

    lax.fori_loop(0, dpt // L, dgrp, None)
    pltpu.sync_copy(de_v, ed_hbm.at[pl.ds(dbase, dpt)])


def _edge_body(x_hbm, e3_hbm, ed_hbm,
               fb_hbm, pc_hbm,
               eidx0_v, eidx1_v, gsel0_v, ssel0_v, gsel1_v, ssel1_v,
               w0_v, w1_v, rows0_v, rows1_v,
               ed_v, sca_v, zrows_v, sem0, sem1, acc_sh):
    cid = lax.axis_index("c")
    sid = lax.axis_index("s")
    is_f = cid == 0
    lane = lax.iota(jnp.int32, L)

    pltpu.sync_copy(ed_hbm, ed_v)

    def zrow(j, _):
        zr = zrows_v.at[j]
        for r in range(D // L):
            zr[pl.ds(r * L, L)] = jnp.zeros((L,), jnp.float32)
        return _

    lax.fori_loop(0, 64, zrow, None)
    for k in range(RRT // 64):
        pltpu.sync_copy(zrows_v, acc_sh.at[pl.ds(sid * RRT + k * 64, 64)])

    def zsca(g, _):
        sca_v[pl.ds(g * L, L)] = jnp.zeros((L,), jnp.float32)
        return _

    lax.fori_loop(0, NP // L, zsca, None)
    plsc.subcore_barrier()

    ebase = sid * EPT

    def prefetch(ck, eidx_v, gsel_v, ssel_v, w_v, rows_v, sem):
        pltpu.sync_copy(e3_hbm.at[pl.ds(ck * (3 * C), 3 * C)], eidx_v)

        def sel(g, _):
            sl = pl.ds(g * L, L)
            a = eidx_v[pl.ds(g * L, L)]
            b = eidx_v[pl.ds(C + g * L, L)]
            gsel_v[sl] = jnp.where(is_f, a, b)
            ssel_v[sl] = jnp.where(is_f, b, a)
            edl = plsc.load_gather(ed_v, [eidx_v[pl.ds(2 * C + g * L, L)]])
            w_v[sl] = jnp.where(is_f, edl, jnp.ones((L,), jnp.float32))
            return _

        lax.fori_loop(0, C // L, sel, None)
        pltpu.async_copy(x_hbm.at[gsel_v], rows_v, sem)

    def process(ssel_v, w_v, rows_v):
        def scale(g, _):
            wl = w_v[pl.ds(g * L, L)]
            for jj in range(L):
                j = g * L + jj
                w = _bcast_lane(wl, jj)
                rr = rows_v.at[j]
                for r in range(D // L):
                    sl = pl.ds(r * L, L)
                    rr[sl] = rr[sl] * w
            return _

        lax.fori_loop(0, C // L, scale, None)
        pltpu.sync_copy(rows_v, acc_sh.at[ssel_v], add=True)

        def sacc(g, _):
            sl = pl.ds(g * L, L)
            plsc.addupdate_scatter(sca_v, [ssel_v[sl]], w_v[sl])
            return _

        lax.fori_loop(0, C // L, sacc, None)

    cbase = sid * NCHUNK
    prefetch(cbase, eidx0_v, gsel0_v, ssel0_v, w0_v, rows0_v, sem0)

    def pair(g, _):
        prefetch(cbase + 2 * g + 1,
                 eidx1_v, gsel1_v, ssel1_v, w1_v, rows1_v, sem1)
        pltpu.make_async_copy(x_hbm.at[gsel0_v], rows0_v, sem0).wait()
        process(ssel0_v, w0_v, rows0_v)
        ck2 = jnp.minimum(cbase + 2 * g + 2, E // C - 1)
        prefetch(ck2, eidx0_v, gsel0_v, ssel0_v, w0_v, rows0_v, sem0)
        pltpu.make_async_copy(x_hbm.at[gsel1_v], rows1_v, sem1).wait()
        process(ssel1_v, w1_v, rows1_v)
        return _

    lax.fori_loop(0, NCHUNK // 2, pair, None)
    pltpu.make_async_copy(x_hbm.at[gsel0_v], rows0_v, sem0).wait()
    plsc.subcore_barrier()

    for k in range(RRT // C):
        rb = sid * RRT + k * C
        pltpu.sync_copy(acc_sh.at[pl.ds(rb, C)], rows0_v)
        pltpu.sync_copy(rows0_v, fb_hbm.at[cid, pl.ds(rb, C)])

    pltpu.sync_copy(sca_v, pc_hbm.at[cid, sid])


@functools.partial(
    pl.kernel,
    out_type=(
        jax.ShapeDtypeStruct((NP, D), jnp.float32),
        jax.ShapeDtypeStruct((NP,), jnp.float32),
        jax.ShapeDtypeStruct((NP,), jnp.float32),
        jax.ShapeDtypeStruct((ND,), jnp.float32),
    ),
    mesh=plsc.VectorSubcoreMesh(core_axis_name="c", subcore_axis_name="s"),
    scratch_types=[
        pltpu.VMEM((RC,), jnp.int32),
        pltpu.VMEM((RC, D), jnp.float32),
        pltpu.VMEM((D,), jnp.float32),
        pltpu.VMEM((D,), jnp.float32),
        pltpu.VMEM((RC,), jnp.float32),
        pltpu.VMEM((RC,), jnp.float32),
        pltpu.VMEM((ND // NW, D), jnp.float32),
        pltpu.VMEM((ND // NW,), jnp.float32),
        pltpu.SemaphoreType.DMA,
    ],
)
def _scores_kernel(*refs):
    _scores_body(*refs)


@functools.partial(
    pl.kernel,
    out_type=(
        jax.ShapeDtypeStruct((NC, NP, D), jnp.float32),
        jax.ShapeDtypeStruct((NC, NS, NP), jnp.float32),
    ),
    mesh=plsc.VectorSubcoreMesh(core_axis_name="c", subcore_axis_name="s"),
    scratch_types=[
        pltpu.VMEM((3 * C,), jnp.int32),
        pltpu.VMEM((3 * C,), jnp.int32),
        pltpu.VMEM((C,), jnp.int32),
        pltpu.VMEM((C,), jnp.int32),
        pltpu.VMEM((C,), jnp.int32),
        pltpu.VMEM((C,), jnp.int32),
        pltpu.VMEM((C,), jnp.float32),
        pltpu.VMEM((C,), jnp.float32),
        pltpu.VMEM((C, D), jnp.float32),
        pltpu.VMEM((C, D), jnp.float32),
        pltpu.VMEM((ND,), jnp.float32),
        pltpu.VMEM((NP,), jnp.float32),
        pltpu.VMEM((64, D), jnp.float32),
        pltpu.SemaphoreType.DMA,
        pltpu.SemaphoreType.DMA,
        pltpu.VMEM_SHARED((NP, D), jnp.float32),
    ],
    compiler_params=pltpu.CompilerParams(needs_layout_passes=False),
)
def _edge_kernel(*refs):
    _edge_body(*refs)


def _combine_body(f_ref, b_ref, es_ref, edt_ref, p_ref, c_ref, o_ref):
    es = es_ref[...]
    edt = edt_ref[...]
    p = jnp.sum(p_ref[...], axis=0)[:, None]
    c = jnp.sum(c_ref[...], axis=0)[:, None]
    denom = es * p + edt * c + 1e-16
    o_ref[...] = (es * f_ref[...] + edt * b_ref[...]) / denom


_combine = pl.pallas_call(
    _combine_body,
    grid=(10,),
    in_specs=[
        pl.BlockSpec((NP // 10, D), lambda i: (i, 0)),
        pl.BlockSpec((NP // 10, D), lambda i: (i, 0)),
        pl.BlockSpec((NP // 10, 1), lambda i: (i, 0)),
        pl.BlockSpec((NP // 10, 1), lambda i: (i, 0)),
        pl.BlockSpec((NS, NP // 10), lambda i: (0, i)),
        pl.BlockSpec((NS, NP // 10), lambda i: (0, i)),
    ],
    out_specs=pl.BlockSpec((NP // 10, D), lambda i: (i, 0)),
    out_shape=jax.ShapeDtypeStruct((NP, D), jnp.float32),
)


@jax.jit
def kernel(POI_embs, delta_dis_embs, sess_idx, edge_index, edge_dist,
           attention_weight, alpha_src, alpha_dst):
    w_src = attention_weight.T @ alpha_src
    w_dst = attention_weight.T @ alpha_dst
    sidx = jnp.concatenate(
        [sess_idx, jnp.zeros((NP - N,), jnp.int32)])
    x, es, edt, ed = _scores_kernel(
        POI_embs, sidx, delta_dis_embs, w_src, w_dst)
    e3 = jnp.concatenate(
        [edge_index[0].reshape(E // C, C),
         edge_index[1].reshape(E // C, C),
         edge_dist.reshape(E // C, C)], axis=1).reshape(-1)
    fb, pc = _edge_kernel(x, e3, ed)
    h = _combine(fb[0], fb[1], es[:, None], edt[:, None], pc[0], pc[1])
    return h[:N]


# speedup vs baseline: 51.5084x; 1.0246x over previous
import functools

import jax
import jax.numpy as jnp
from jax import lax
from jax.experimental import pallas as pl
from jax.experimental.pallas import tpu as pltpu
from jax.experimental.pallas import tpu_sc as plsc

NC = 2
NS = 16
NW = NC * NS
L = 16

N = 10000
NP = 10240
D = 128
E = 320000
ND = 1024

RPT = NP // NW


def _vgather(v, idx):
    return lax.gather(
        v, idx[:, None],
        dimension_numbers=lax.GatherDimensionNumbers(
            offset_dims=(), collapsed_slice_dims=(0,), start_index_map=(0,)),
        slice_sizes=(1,),
        mode=lax.GatherScatterMode.PROMISE_IN_BOUNDS)


def _allsum(v):
    idx = lax.iota(jnp.int32, L)
    for sh in (1, 2, 4, 8):
        v = v + _vgather(v, idx ^ sh)
    return v


def _bcast_lane(v, jj):
    return _vgather(v, jnp.full((L,), jj, jnp.int32))
RC = 80
EPT = E // NS
C = 80
NCHUNK = EPT // C
RRT = NP // NS


def _scores_body(poi_hbm, sidx_hbm, delta_hbm, ws_hbm, wd_hbm,
                 x_hbm, es_hbm, edt_hbm, ed_hbm,
                 idx_v, rows_v, ws_v, wd_v, e_v, f_v,
                 drows_v, de_v, sem):
    wid = lax.axis_index("s") * NC + lax.axis_index("c")
    base = wid * RPT
    pltpu.sync_copy(ws_hbm, ws_v)
    pltpu.sync_copy(wd_hbm, wd_v)
    lane = lax.iota(jnp.int32, L)

    def chunk(k, _):
        cb = base + k * RC
        pltpu.sync_copy(sidx_hbm.at[pl.ds(cb, RC)], idx_v)
        pltpu.async_copy(poi_hbm.at[idx_v], rows_v, sem).wait()
        pltpu.sync_copy(rows_v, x_hbm.at[pl.ds(cb, RC)])

        def grp(g, _):
            sres = jnp.zeros((L,), jnp.float32)
            dres = jnp.zeros((L,), jnp.float32)
            for jj in range(L):
                j = g * L + jj
                acc_s = jnp.zeros((L,), jnp.float32)
                acc_d = jnp.zeros((L,), jnp.float32)
                for r in range(D // L):
                    xr = rows_v[j, pl.ds(r * L, L)]
                    acc_s = acc_s + xr * ws_v[pl.ds(r * L, L)]
                    acc_d = acc_d + xr * wd_v[pl.ds(r * L, L)]
                sres = jnp.where(lane == jj, _allsum(acc_s), sres)
                dres = jnp.where(lane == jj, _allsum(acc_d), dres)
            e_v[pl.ds(g * L, L)] = jnp.exp(sres)
            f_v[pl.ds(g * L, L)] = jnp.exp(dres)
            return _

        lax.fori_loop(0, RC // L, grp, None)
        pltpu.sync_copy(e_v, es_hbm.at[pl.ds(cb, RC)])
        pltpu.sync_copy(f_v, edt_hbm.at[pl.ds(cb, RC)])
        return _

    lax.fori_loop(0, RPT // RC, chunk, None)

    dpt = ND // NW
    dbase = wid * dpt
    pltpu.sync_copy(delta_hbm.at[pl.ds(dbase, dpt)], drows_v)

    def dgrp(g, _):
        dres = jnp.zeros((L,), jnp.float32)
        for jj in range(L):
            j = g * L + jj
            acc = jnp.zeros((L,), jnp.float32)
            for r in range(D // L):
                acc = acc + drows_v[j, pl.ds(r * L, L)] * ws_v[pl.ds(r * L, L)]
            dres = jnp.where(lane == jj, _allsum(acc), dres)
        de_v[pl.ds(g * L, L)] = jnp.exp(dres)
        return _

    lax.fori_loop(0, dpt // L, dgrp, None)
    pltpu.sync_copy(de_v, ed_hbm.at[pl.ds(dbase, dpt)])


def _edge_body(x_hbm, e3_hbm, ed_hbm,
               fb_hbm, pc_hbm,
               eidx0_v, eidx1_v, gsel0_v, ssel0_v, gsel1_v, ssel1_v,
               w0_v, w1_v, rows0_v, rows1_v,
               ed_v, sca_v, zrows_v, sem0, sem1, ssem0, ssem1, acc_sh):
    cid = lax.axis_index("c")
    sid = lax.axis_index("s")
    is_f = cid == 0
    lane = lax.iota(jnp.int32, L)

    pltpu.sync_copy(ed_hbm, ed_v)

    def zrow(j, _):
        zr = zrows_v.at[j]
        for r in range(D // L):
            zr[pl.ds(r * L, L)] = jnp.zeros((L,), jnp.float32)
        return _

    lax.fori_loop(0, 64, zrow, None)
    for k in range(RRT // 64):
        pltpu.sync_copy(zrows_v, acc_sh.at[pl.ds(sid * RRT + k * 64, 64)])

    def zsca(g, _):
        sca_v[pl.ds(g * L, L)] = jnp.zeros((L,), jnp.float32)
        return _

    lax.fori_loop(0, NP // L, zsca, None)
    plsc.subcore_barrier()

    ebase = sid * EPT

    def prefetch(ck, eidx_v, gsel_v, ssel_v, w_v, rows_v, sem):
        pltpu.sync_copy(e3_hbm.at[pl.ds(ck * (3 * C), 3 * C)], eidx_v)

        def sel(g, _):
            sl = pl.ds(g * L, L)
            a = eidx_v[pl.ds(g * L, L)]
            b = eidx_v[pl.ds(C + g * L, L)]
            gsel_v[sl] = jnp.where(is_f, a, b)
            ssel_v[sl] = jnp.where(is_f, b, a)
            edl = plsc.load_gather(ed_v, [eidx_v[pl.ds(2 * C + g * L, L)]])
            w_v[sl] = jnp.where(is_f, edl, jnp.ones((L,), jnp.float32))
            return _

        lax.fori_loop(0, C // L, sel, None)
        pltpu.async_copy(x_hbm.at[gsel_v], rows_v, sem)

    def process(ssel_v, w_v, rows_v, ssem):
        def scale(g, _):
            wl = w_v[pl.ds(g * L, L)]
            for jj in range(L):
                j = g * L + jj
                w = _bcast_lane(wl, jj)
                rr = rows_v.at[j]
                for r in range(D // L):
                    sl = pl.ds(r * L, L)
                    rr[sl] = rr[sl] * w
            return _

        lax.fori_loop(0, C // L, scale, None)
        pltpu.async_copy(rows_v, acc_sh.at[ssel_v], ssem, add=True)

        def sacc(g, _):
            sl = pl.ds(g * L, L)
            plsc.addupdate_scatter(sca_v, [ssel_v[sl]], w_v[sl])
            return _

        lax.fori_loop(0, C // L, sacc, None)

    def wait_scatter(ssel_v, rows_v, ssem):
        pltpu.make_async_copy(rows_v, acc_sh.at[ssel_v], ssem).wait()

    cbase = sid * NCHUNK
    def selpad(g, _):
        ssel1_v[pl.ds(g * L, L)] = jnp.full((L,), NP - 1, jnp.int32)
        return _

    lax.fori_loop(0, C // L, selpad, None)
    pltpu.async_copy(rows1_v, acc_sh.at[ssel1_v], ssem1, add=True)
    prefetch(cbase, eidx0_v, gsel0_v, ssel0_v, w0_v, rows0_v, sem0)

    def pair(g, _):
        wait_scatter(ssel1_v, rows1_v, ssem1)
        prefetch(cbase + 2 * g + 1,
                 eidx1_v, gsel1_v, ssel1_v, w1_v, rows1_v, sem1)
        pltpu.make_async_copy(x_hbm.at[gsel0_v], rows0_v, sem0).wait()
        process(ssel0_v, w0_v, rows0_v, ssem0)
        pltpu.make_async_copy(x_hbm.at[gsel1_v], rows1_v, sem1).wait()
        process(ssel1_v, w1_v, rows1_v, ssem1)
        wait_scatter(ssel0_v, rows0_v, ssem0)
        ck2 = jnp.minimum(cbase + 2 * g + 2, E // C - 1)
        prefetch(ck2, eidx0_v, gsel0_v, ssel0_v, w0_v, rows0_v, sem0)
        return _

    lax.fori_loop(0, NCHUNK // 2, pair, None)
    wait_scatter(ssel1_v, rows1_v, ssem1)
    pltpu.make_async_copy(x_hbm.at[gsel0_v], rows0_v, sem0).wait()
    plsc.subcore_barrier()

    for k in range(RRT // C):
        rb = sid * RRT + k * C
        pltpu.sync_copy(acc_sh.at[pl.ds(rb, C)], rows0_v)
        pltpu.sync_copy(rows0_v, fb_hbm.at[cid, pl.ds(rb, C)])

    pltpu.sync_copy(sca_v, pc_hbm.at[cid, sid])


@functools.partial(
    pl.kernel,
    out_type=(
        jax.ShapeDtypeStruct((NP, D), jnp.float32),
        jax.ShapeDtypeStruct((NP,), jnp.float32),
        jax.ShapeDtypeStruct((NP,), jnp.float32),
        jax.ShapeDtypeStruct((ND,), jnp.float32),
    ),
    mesh=plsc.VectorSubcoreMesh(core_axis_name="c", subcore_axis_name="s"),
    scratch_types=[
        pltpu.VMEM((RC,), jnp.int32),
        pltpu.VMEM((RC, D), jnp.float32),
        pltpu.VMEM((D,), jnp.float32),
        pltpu.VMEM((D,), jnp.float32),
        pltpu.VMEM((RC,), jnp.float32),
        pltpu.VMEM((RC,), jnp.float32),
        pltpu.VMEM((ND // NW, D), jnp.float32),
        pltpu.VMEM((ND // NW,), jnp.float32),
        pltpu.SemaphoreType.DMA,
    ],
)
def _scores_kernel(*refs):
    _scores_body(*refs)


@functools.partial(
    pl.kernel,
    out_type=(
        jax.ShapeDtypeStruct((NC, NP, D), jnp.float32),
        jax.ShapeDtypeStruct((NC, NS, NP), jnp.float32),
    ),
    mesh=plsc.VectorSubcoreMesh(core_axis_name="c", subcore_axis_name="s"),
    scratch_types=[
        pltpu.VMEM((3 * C,), jnp.int32),
        pltpu.VMEM((3 * C,), jnp.int32),
        pltpu.VMEM((C,), jnp.int32),
        pltpu.VMEM((C,), jnp.int32),
        pltpu.VMEM((C,), jnp.int32),
        pltpu.VMEM((C,), jnp.int32),
        pltpu.VMEM((C,), jnp.float32),
        pltpu.VMEM((C,), jnp.float32),
        pltpu.VMEM((C, D), jnp.float32),
        pltpu.VMEM((C, D), jnp.float32),
        pltpu.VMEM((ND,), jnp.float32),
        pltpu.VMEM((NP,), jnp.float32),
        pltpu.VMEM((64, D), jnp.float32),
        pltpu.SemaphoreType.DMA,
        pltpu.SemaphoreType.DMA,
        pltpu.SemaphoreType.DMA,
        pltpu.SemaphoreType.DMA,
        pltpu.VMEM_SHARED((NP, D), jnp.float32),
    ],
    compiler_params=pltpu.CompilerParams(needs_layout_passes=False),
)
def _edge_kernel(*refs):
    _edge_body(*refs)


def _combine_body(f_ref, b_ref, es_ref, edt_ref, p_ref, c_ref, o_ref):
    es = es_ref[...]
    edt = edt_ref[...]
    p = jnp.sum(p_ref[...], axis=0)[:, None]
    c = jnp.sum(c_ref[...], axis=0)[:, None]
    denom = es * p + edt * c + 1e-16
    o_ref[...] = (es * f_ref[...] + edt * b_ref[...]) / denom


_combine = pl.pallas_call(
    _combine_body,
    grid=(10,),
    in_specs=[
        pl.BlockSpec((NP // 10, D), lambda i: (i, 0)),
        pl.BlockSpec((NP // 10, D), lambda i: (i, 0)),
        pl.BlockSpec((NP // 10, 1), lambda i: (i, 0)),
        pl.BlockSpec((NP // 10, 1), lambda i: (i, 0)),
        pl.BlockSpec((NS, NP // 10), lambda i: (0, i)),
        pl.BlockSpec((NS, NP // 10), lambda i: (0, i)),
    ],
    out_specs=pl.BlockSpec((NP // 10, D), lambda i: (i, 0)),
    out_shape=jax.ShapeDtypeStruct((NP, D), jnp.float32),
)


@jax.jit
def kernel(POI_embs, delta_dis_embs, sess_idx, edge_index, edge_dist,
           attention_weight, alpha_src, alpha_dst):
    w_src = attention_weight.T @ alpha_src
    w_dst = attention_weight.T @ alpha_dst
    sidx = jnp.concatenate(
        [sess_idx, jnp.zeros((NP - N,), jnp.int32)])
    x, es, edt, ed = _scores_kernel(
        POI_embs, sidx, delta_dis_embs, w_src, w_dst)
    e3 = jnp.concatenate(
        [edge_index[0].reshape(E // C, C),
         edge_index[1].reshape(E // C, C),
         edge_dist.reshape(E // C, C)], axis=1).reshape(-1)
    fb, pc = _edge_kernel(x, e3, ed)
    h = _combine(fb[0], fb[1], es[:, None], edt[:, None], pc[0], pc[1])
    return h[:N]


# speedup vs baseline: 59.4445x; 1.1541x over previous
import functools

import jax
import jax.numpy as jnp
from jax import lax
from jax.experimental import pallas as pl
from jax.experimental.pallas import tpu as pltpu
from jax.experimental.pallas import tpu_sc as plsc

NC = 2
NS = 16
NW = NC * NS
L = 16

N = 10000
NP = 10240
D = 128
E = 320000
ND = 1024

RPT = NP // NW


def _vgather(v, idx):
    return lax.gather(
        v, idx[:, None],
        dimension_numbers=lax.GatherDimensionNumbers(
            offset_dims=(), collapsed_slice_dims=(0,), start_index_map=(0,)),
        slice_sizes=(1,),
        mode=lax.GatherScatterMode.PROMISE_IN_BOUNDS)


def _allsum(v):
    idx = lax.iota(jnp.int32, L)
    for sh in (1, 2, 4, 8):
        v = v + _vgather(v, idx ^ sh)
    return v


def _bcast_lane(v, jj):
    return _vgather(v, jnp.full((L,), jj, jnp.int32))
RC = 80
EPT = E // NS
C = 80
NCHUNK = EPT // C
RRT = NP // NS


def _scores_body(poi_hbm, sidx_hbm, delta_hbm, ws_hbm, wd_hbm,
                 x_hbm, es_hbm, edt_hbm, ed_hbm,
                 idx_v, rows_v, ws_v, wd_v, e_v, f_v,
                 drows_v, de_v, sem):
    wid = lax.axis_index("s") * NC + lax.axis_index("c")
    base = wid * RPT
    pltpu.sync_copy(ws_hbm, ws_v)
    pltpu.sync_copy(wd_hbm, wd_v)
    lane = lax.iota(jnp.int32, L)

    def chunk(k, _):
        cb = base + k * RC
        pltpu.sync_copy(sidx_hbm.at[pl.ds(cb, RC)], idx_v)
        pltpu.async_copy(poi_hbm.at[idx_v], rows_v, sem).wait()
        pltpu.sync_copy(rows_v, x_hbm.at[pl.ds(cb, RC)])

        def grp(g, _):
            sres = jnp.zeros((L,), jnp.float32)
            dres = jnp.zeros((L,), jnp.float32)
            for jj in range(L):
                j = g * L + jj
                acc_s = jnp.zeros((L,), jnp.float32)
                acc_d = jnp.zeros((L,), jnp.float32)
                for r in range(D // L):
                    xr = rows_v[j, pl.ds(r * L, L)]
                    acc_s = acc_s + xr * ws_v[pl.ds(r * L, L)]
                    acc_d = acc_d + xr * wd_v[pl.ds(r * L, L)]
                sres = jnp.where(lane == jj, _allsum(acc_s), sres)
                dres = jnp.where(lane == jj, _allsum(acc_d), dres)
            e_v[pl.ds(g * L, L)] = jnp.exp(sres)
            f_v[pl.ds(g * L, L)] = jnp.exp(dres)
            return _

        lax.fori_loop(0, RC // L, grp, None)
        pltpu.sync_copy(e_v, es_hbm.at[pl.ds(cb, RC)])
        pltpu.sync_copy(f_v, edt_hbm.at[pl.ds(cb, RC)])
        return _

    lax.fori_loop(0, RPT // RC, chunk, None)

    dpt = ND // NW
    dbase = wid * dpt
    pltpu.sync_copy(delta_hbm.at[pl.ds(dbase, dpt)], drows_v)

    def dgrp(g, _):
        dres = jnp.zeros((L,), jnp.float32)
        for jj in range(L):
            j = g * L + jj
            acc = jnp.zeros((L,), jnp.float32)
            for r in range(D // L):
                acc = acc + drows_v[j, pl.ds(r * L, L)] * ws_v[pl.ds(r * L, L)]
            dres = jnp.where(lane == jj, _allsum(acc), dres)
        de_v[pl.ds(g * L, L)] = jnp.exp(dres)
        return _

    lax.fori_loop(0, dpt // L, dgrp, None)
    pltpu.sync_copy(de_v, ed_hbm.at[pl.ds(dbase, dpt)])


def _edge_body(x_hbm, e3_hbm, ed_hbm,
               fb_hbm, pc_hbm,
               eidx0_v, eidx1_v, gsel0_v, ssel0_v, gsel1_v, ssel1_v,
               w0_v, w1_v, rows0_v, rows1_v,
               ed_v, sca_v, zrows_v, sem0, sem1, ssem0, ssem1,
               isem0, isem1, acc_sh):
    cid = lax.axis_index("c")
    sid = lax.axis_index("s")
    is_f = cid == 0
    lane = lax.iota(jnp.int32, L)

    pltpu.sync_copy(ed_hbm, ed_v)

    def zrow(j, _):
        zr = zrows_v.at[j]
        for r in range(D // L):
            zr[pl.ds(r * L, L)] = jnp.zeros((L,), jnp.float32)
        return _

    lax.fori_loop(0, 64, zrow, None)
    for k in range(RRT // 64):
        pltpu.sync_copy(zrows_v, acc_sh.at[pl.ds(sid * RRT + k * 64, 64)])

    def zsca(g, _):
        sca_v[pl.ds(g * L, L)] = jnp.zeros((L,), jnp.float32)
        return _

    lax.fori_loop(0, NP // L, zsca, None)
    plsc.subcore_barrier()

    ebase = sid * EPT

    def fire_idx(ck, eidx_v, isem):
        pltpu.async_copy(e3_hbm.at[pl.ds(ck * (3 * C), 3 * C)], eidx_v, isem)

    def wait_idx(ck, eidx_v, isem):
        pltpu.make_async_copy(
            e3_hbm.at[pl.ds(ck * (3 * C), 3 * C)], eidx_v, isem).wait()

    def sel_and_gather(eidx_v, gsel_v, ssel_v, w_v, rows_v, sem):
        def sel(g, _):
            sl = pl.ds(g * L, L)
            a = eidx_v[pl.ds(g * L, L)]
            b = eidx_v[pl.ds(C + g * L, L)]
            gsel_v[sl] = jnp.where(is_f, a, b)
            ssel_v[sl] = jnp.where(is_f, b, a)
            edl = plsc.load_gather(ed_v, [eidx_v[pl.ds(2 * C + g * L, L)]])
            w_v[sl] = jnp.where(is_f, edl, jnp.ones((L,), jnp.float32))
            return _

        lax.fori_loop(0, C // L, sel, None)
        pltpu.async_copy(x_hbm.at[gsel_v], rows_v, sem)

    def process(ssel_v, w_v, rows_v, ssem):
        def scale(g, _):
            wl = w_v[pl.ds(g * L, L)]
            for jj in range(L):
                j = g * L + jj
                w = _bcast_lane(wl, jj)
                rr = rows_v.at[j]
                for r in range(D // L):
                    sl = pl.ds(r * L, L)
                    rr[sl] = rr[sl] * w
            return _

        lax.fori_loop(0, C // L, scale, None)
        pltpu.async_copy(rows_v, acc_sh.at[ssel_v], ssem, add=True)

        def sacc(g, _):
            sl = pl.ds(g * L, L)
            plsc.addupdate_scatter(sca_v, [ssel_v[sl]], w_v[sl])
            return _

        lax.fori_loop(0, C // L, sacc, None)

    def wait_scatter(ssel_v, rows_v, ssem):
        pltpu.make_async_copy(rows_v, acc_sh.at[ssel_v], ssem).wait()

    cbase = sid * NCHUNK
    lastck = E // C - 1
    def selpad(g, _):
        ssel1_v[pl.ds(g * L, L)] = jnp.full((L,), NP - 1, jnp.int32)
        return _

    lax.fori_loop(0, C // L, selpad, None)
    pltpu.async_copy(rows1_v, acc_sh.at[ssel1_v], ssem1, add=True)
    fire_idx(cbase, eidx0_v, isem0)
    wait_idx(cbase, eidx0_v, isem0)
    sel_and_gather(eidx0_v, gsel0_v, ssel0_v, w0_v, rows0_v, sem0)
    fire_idx(cbase + 1, eidx1_v, isem1)

    def pair(g, _):
        wait_scatter(ssel1_v, rows1_v, ssem1)
        ck1 = cbase + 2 * g + 1
        wait_idx(ck1, eidx1_v, isem1)
        sel_and_gather(eidx1_v, gsel1_v, ssel1_v, w1_v, rows1_v, sem1)
        ck2 = jnp.minimum(cbase + 2 * g + 2, lastck)
        fire_idx(ck2, eidx0_v, isem0)
        pltpu.make_async_copy(x_hbm.at[gsel0_v], rows0_v, sem0).wait()
        process(ssel0_v, w0_v, rows0_v, ssem0)
        pltpu.make_async_copy(x_hbm.at[gsel1_v], rows1_v, sem1).wait()
        process(ssel1_v, w1_v, rows1_v, ssem1)
        wait_scatter(ssel0_v, rows0_v, ssem0)
        wait_idx(ck2, eidx0_v, isem0)
        sel_and_gather(eidx0_v, gsel0_v, ssel0_v, w0_v, rows0_v, sem0)
        ck3 = jnp.minimum(cbase + 2 * g + 3, lastck)
        fire_idx(ck3, eidx1_v, isem1)
        return _

    lax.fori_loop(0, NCHUNK // 2, pair, None)
    wait_scatter(ssel1_v, rows1_v, ssem1)
    pltpu.make_async_copy(x_hbm.at[gsel0_v], rows0_v, sem0).wait()
    wait_idx(lastck, eidx1_v, isem1)
    plsc.subcore_barrier()

    for k in range(RRT // C):
        rb = sid * RRT + k * C
        pltpu.sync_copy(acc_sh.at[pl.ds(rb, C)], rows0_v)
        pltpu.sync_copy(rows0_v, fb_hbm.at[cid, pl.ds(rb, C)])

    pltpu.sync_copy(sca_v, pc_hbm.at[cid, sid])


@functools.partial(
    pl.kernel,
    out_type=(
        jax.ShapeDtypeStruct((NP, D), jnp.float32),
        jax.ShapeDtypeStruct((NP,), jnp.float32),
        jax.ShapeDtypeStruct((NP,), jnp.float32),
        jax.ShapeDtypeStruct((ND,), jnp.float32),
    ),
    mesh=plsc.VectorSubcoreMesh(core_axis_name="c", subcore_axis_name="s"),
    scratch_types=[
        pltpu.VMEM((RC,), jnp.int32),
        pltpu.VMEM((RC, D), jnp.float32),
        pltpu.VMEM((D,), jnp.float32),
        pltpu.VMEM((D,), jnp.float32),
        pltpu.VMEM((RC,), jnp.float32),
        pltpu.VMEM((RC,), jnp.float32),
        pltpu.VMEM((ND // NW, D), jnp.float32),
        pltpu.VMEM((ND // NW,), jnp.float32),
        pltpu.SemaphoreType.DMA,
    ],
)
def _scores_kernel(*refs):
    _scores_body(*refs)


@functools.partial(
    pl.kernel,
    out_type=(
        jax.ShapeDtypeStruct((NC, NP, D), jnp.float32),
        jax.ShapeDtypeStruct((NC, NS, NP), jnp.float32),
    ),
    mesh=plsc.VectorSubcoreMesh(core_axis_name="c", subcore_axis_name="s"),
    scratch_types=[
        pltpu.VMEM((3 * C,), jnp.int32),
        pltpu.VMEM((3 * C,), jnp.int32),
        pltpu.VMEM((C,), jnp.int32),
        pltpu.VMEM((C,), jnp.int32),
        pltpu.VMEM((C,), jnp.int32),
        pltpu.VMEM((C,), jnp.int32),
        pltpu.VMEM((C,), jnp.float32),
        pltpu.VMEM((C,), jnp.float32),
        pltpu.VMEM((C, D), jnp.float32),
        pltpu.VMEM((C, D), jnp.float32),
        pltpu.VMEM((ND,), jnp.float32),
        pltpu.VMEM((NP,), jnp.float32),
        pltpu.VMEM((64, D), jnp.float32),
        pltpu.SemaphoreType.DMA,
        pltpu.SemaphoreType.DMA,
        pltpu.SemaphoreType.DMA,
        pltpu.SemaphoreType.DMA,
        pltpu.SemaphoreType.DMA,
        pltpu.SemaphoreType.DMA,
        pltpu.VMEM_SHARED((NP, D), jnp.float32),
    ],
    compiler_params=pltpu.CompilerParams(needs_layout_passes=False),
)
def _edge_kernel(*refs):
    _edge_body(*refs)


def _combine_body(f_ref, b_ref, es_ref, edt_ref, p_ref, c_ref, o_ref):
    es = es_ref[...]
    edt = edt_ref[...]
    p = jnp.sum(p_ref[...], axis=0)[:, None]
    c = jnp.sum(c_ref[...], axis=0)[:, None]
    denom = es * p + edt * c + 1e-16
    o_ref[...] = (es * f_ref[...] + edt * b_ref[...]) / denom


_combine = pl.pallas_call(
    _combine_body,
    grid=(10,),
    in_specs=[
        pl.BlockSpec((NP // 10, D), lambda i: (i, 0)),
        pl.BlockSpec((NP // 10, D), lambda i: (i, 0)),
        pl.BlockSpec((NP // 10, 1), lambda i: (i, 0)),
        pl.BlockSpec((NP // 10, 1), lambda i: (i, 0)),
        pl.BlockSpec((NS, NP // 10), lambda i: (0, i)),
        pl.BlockSpec((NS, NP // 10), lambda i: (0, i)),
    ],
    out_specs=pl.BlockSpec((NP // 10, D), lambda i: (i, 0)),
    out_shape=jax.ShapeDtypeStruct((NP, D), jnp.float32),
)


@jax.jit
def kernel(POI_embs, delta_dis_embs, sess_idx, edge_index, edge_dist,
           attention_weight, alpha_src, alpha_dst):
    w_src = attention_weight.T @ alpha_src
    w_dst = attention_weight.T @ alpha_dst
    sidx = jnp.concatenate(
        [sess_idx, jnp.zeros((NP - N,), jnp.int32)])
    x, es, edt, ed = _scores_kernel(
        POI_embs, sidx, delta_dis_embs, w_src, w_dst)
    e3 = jnp.concatenate(
        [edge_index[0].reshape(E // C, C),
         edge_index[1].reshape(E // C, C),
         edge_dist.reshape(E // C, C)], axis=1).reshape(-1)
    fb, pc = _edge_kernel(x, e3, ed)
    h = _combine(fb[0], fb[1], es[:, None], edt[:, None], pc[0], pc[1])
    return h[:N]
